# baseline (speedup 1.0000x reference)
import functools

import jax
import jax.numpy as jnp
from jax import lax
from jax.experimental import pallas as pl
from jax.experimental.pallas import tpu as pltpu
from jax.experimental.pallas import tpu_sc as plsc

_EMB = 32
_HIDDEN = 64
_STATE = 32

_NC = 2
_NS = 16
_NW = _NC * _NS

_IDX_ROWS = 32


def _sc_gather(table, idx_flat_pad, b, l, lp):
    n_total = b * l
    rows_per_w = b // _NW
    n_chunks = rows_per_w // _IDX_ROWS
    chunk = _IDX_ROWS * l
    mesh = plsc.VectorSubcoreMesh(core_axis_name="c", subcore_axis_name="s")

    @functools.partial(
        pl.kernel,
        mesh=mesh,
        compiler_params=pltpu.CompilerParams(use_tc_tiling_on_sc=False),
        out_type=jax.ShapeDtypeStruct((n_total, _EMB), jnp.float32),
        scratch_types=[
            pltpu.VMEM((_IDX_ROWS * lp,), jnp.int32),
            pltpu.VMEM((chunk, _EMB), jnp.float32),
            pltpu.SemaphoreType.DMA,
        ],
    )
    def gather_kernel(table_hbm, idx_hbm, out_hbm, idx_v, rows_v, sem):
        wid = lax.axis_index("s") * _NC + lax.axis_index("c")
        row_base = wid * rows_per_w

        def body(i, _):
            row0 = row_base + i * _IDX_ROWS
            pltpu.sync_copy(
                idx_hbm.at[pl.ds(row0 * lp, _IDX_ROWS * lp)], idx_v)

            def fire(r, _):
                pltpu.async_copy(
                    table_hbm.at[idx_v.at[pl.ds(r * lp, l)]],
                    rows_v.at[pl.ds(r * l, l)],
                    sem,
                )
                return 0

            lax.fori_loop(0, _IDX_ROWS, fire, 0)

            def drain(r, _):
                pltpu.make_async_copy(
                    table_hbm.at[idx_v.at[pl.ds(r * lp, l)]],
                    rows_v.at[pl.ds(r * l, l)],
                    sem,
                ).wait()
                return 0

            lax.fori_loop(0, _IDX_ROWS, drain, 0)
            pltpu.sync_copy(rows_v, out_hbm.at[pl.ds(row0 * l, chunk)])
            return 0

        lax.fori_loop(0, n_chunks, body, 0)

    return gather_kernel(table, idx_flat_pad)


def _pad_idx_body(i_ref, o_ref):
    o_ref[:, pl.ds(0, i_ref.shape[1])] = i_ref[...]


def _tc_pad_idx(cards_id, bb=2048):
    b, l = cards_id.shape
    return pl.pallas_call(
        _pad_idx_body,
        grid=(b // bb,),
        in_specs=[pl.BlockSpec((bb, l), lambda i: (i, 0))],
        out_specs=pl.BlockSpec((bb, 128), lambda i: (i, 0)),
        out_shape=jax.ShapeDtypeStruct((b, 128), jnp.int32),
    )(cards_id)


def _mlp_body(x4_ref, w1_ref, b1_ref, w2_ref, b2_ref, o_ref):
    x4 = x4_ref[...]
    xs = [x4[:, 32 * k:32 * k + 32] for k in range(4)]
    x = jnp.stack(xs, axis=1).reshape(x4.shape[0] * 4, _EMB)
    h = jnp.dot(x, w1_ref[...], preferred_element_type=jnp.float32) + b1_ref[...]
    h = 1.0 / (1.0 + jnp.exp(-h))
    y = jnp.dot(h, w2_ref[...], preferred_element_type=jnp.float32) + b2_ref[...]
    y = 1.0 / (1.0 + jnp.exp(-y))
    o_ref[...] = y.reshape(o_ref.shape)


def _tc_mlp(x, w1, b1, w2, b2, b, l, bb):
    grid = (b // bb,)
    return pl.pallas_call(
        _mlp_body,
        grid=grid,
        in_specs=[
            pl.BlockSpec((bb * l // 4, 4 * _EMB), lambda i: (i, 0)),
            pl.BlockSpec((_EMB, _HIDDEN), lambda i: (0, 0)),
            pl.BlockSpec((1, _HIDDEN), lambda i: (0, 0)),
            pl.BlockSpec((_HIDDEN, _STATE), lambda i: (0, 0)),
            pl.BlockSpec((1, _STATE), lambda i: (0, 0)),
        ],
        out_specs=pl.BlockSpec((bb, l, _STATE), lambda i: (i, 0, 0)),
        out_shape=jax.ShapeDtypeStruct((b, l, _STATE), jnp.float32),
    )(x, w1, b1.reshape(1, _HIDDEN), w2, b2.reshape(1, _STATE))


def kernel(cards_id, emb_table, W1, b1, W2, b2):
    b, l = cards_id.shape
    idx_pad = _tc_pad_idx(cards_id.astype(jnp.int32))
    gathered = _sc_gather(emb_table, idx_pad.reshape(-1), b, l, 128)
    x4 = gathered.reshape(b * l // 4, 4 * _EMB)
    return _tc_mlp(x4, W1, b1, W2, b2, b, l, bb=128)
